# baseline (speedup 1.0000x reference)
import functools

import jax
import jax.numpy as jnp
from jax import lax
from jax.experimental import pallas as pl
from jax.experimental.pallas import tpu as pltpu
from jax.experimental.pallas import tpu_sc as plsc

VOCAB = 1000000
D = 64
BATCH = 16384
HIST = 50
B = BATCH * HIST

_info = plsc.get_sparse_core_info()
NC, NS = _info.num_cores, _info.num_subcores
NW = NC * NS
ROWS_PER_W = BATCH // NW
B_PER_W = B // NW
BB_PER_W = ROWS_PER_W // 128
N_ROUNDS = HIST * BB_PER_W


@functools.partial(
    pl.kernel,
    mesh=plsc.VectorSubcoreMesh(core_axis_name="c", subcore_axis_name="s"),
    out_type=jax.ShapeDtypeStruct((HIST, 8, BATCH // 128, 8, 128), jnp.float32),
    scratch_types=[
        pltpu.VMEM((B_PER_W,), jnp.int32),
        pltpu.VMEM((HIST, BB_PER_W, 128), jnp.int32),
        [pltpu.VMEM((128, D), jnp.float32) for _ in range(2)],
        [pltpu.VMEM((D, 128), jnp.float32) for _ in range(2)],
        [pltpu.SemaphoreType.DMA for _ in range(2)],
        [pltpu.SemaphoreType.DMA for _ in range(2)],
    ],
    compiler_params=pltpu.CompilerParams(use_tc_tiling_on_sc=False, needs_layout_passes=False),
)
def _gather_kernel(table_hbm, idx_hbm, out_hbm, idx_all, idx_t, rows, tbuf, sg, sw):
    wid = lax.axis_index("s") * NC + lax.axis_index("c")
    base = wid * B_PER_W
    pltpu.sync_copy(idx_hbm.at[pl.ds(base, B_PER_W)], idx_all)

    lane = lax.iota(jnp.int32, 16)
    lane50 = lane * HIST

    def idx_t_body(h, carry):
        for g in range(2 * NS):
            v = plsc.load_gather(idx_all, [lane50 + (g * 16 * HIST + h)])
            idx_t[h, g // 8, pl.ds((g % 8) * 16, 16)] = v
        return carry

    lax.fori_loop(0, HIST, idx_t_body, 0)

    def fire_gather(r, j):
        pltpu.async_copy(
            table_hbm.at[idx_t.at[r // BB_PER_W, r % BB_PER_W]], rows[j], sg[j]
        )

    def wait_gather(j):
        pltpu.make_async_copy(
            table_hbm.at[pl.ds(0, 128)], rows[j], sg[j]
        ).wait()

    def transpose_rows(j):
        def tbody(b, carry):
            for g in range(D // 16):
                v = rows[j][b, pl.ds(g * 16, 16)]
                plsc.store_scatter(
                    tbuf[j], [lane + g * 16, jnp.full((16,), b, jnp.int32)], v
                )
            return carry

        lax.fori_loop(0, 128, tbody, 0)

    def fire_write(r, j):
        h = r // BB_PER_W
        bbg = wid * BB_PER_W + r % BB_PER_W
        for db in range(8):
            pltpu.async_copy(
                tbuf[j].at[pl.ds(db * 8, 8)], out_hbm.at[h, db, bbg], sw[j]
            )

    def wait_write(j):
        for db in range(8):
            pltpu.make_async_copy(
                tbuf[j].at[pl.ds(0, 8)], out_hbm.at[0, 0, 0], sw[j]
            ).wait()

    fire_gather(0, 0)
    wait_gather(0)
    fire_gather(1, 1)
    transpose_rows(0)
    fire_write(0, 0)
    wait_gather(1)
    fire_gather(2, 0)
    transpose_rows(1)
    fire_write(1, 1)

    def body(i, carry):
        r = 2 * i
        wait_gather(0)
        fire_gather(r + 1, 1)
        wait_write(0)
        transpose_rows(0)
        fire_write(r, 0)
        wait_gather(1)
        fire_gather(r + 2, 0)
        wait_write(1)
        transpose_rows(1)
        fire_write(r + 1, 1)
        return carry

    lax.fori_loop(1, N_ROUNDS // 2 - 1, body, 0)

    rl = N_ROUNDS - 2
    wait_gather(0)
    fire_gather(rl + 1, 1)
    wait_write(0)
    transpose_rows(0)
    fire_write(rl, 0)
    wait_gather(1)
    wait_write(1)
    transpose_rows(1)
    fire_write(rl + 1, 1)

    wait_write(0)
    wait_write(1)


def kernel(x, table):
    idx = x.reshape(-1).astype(jnp.int32)
    o5 = _gather_kernel(table, idx)
    return o5.transpose(2, 4, 0, 1, 3).reshape(BATCH, HIST, D)
